# baseline (speedup 1.0000x reference)
import functools

import jax
import jax.numpy as jnp
from jax import lax
from jax.experimental import pallas as pl
from jax.experimental.pallas import tpu as pltpu
from jax.experimental.pallas import tpu_sc as plsc

C = 32
V_IN = 163842
V_OUT = 40962
B = 2
EPS = 1e-5
NEG = 0.2

R = 1024
VP = 43008
NB = VP // R
NP = B * VP
NPB = NP // R
VIP = 165888
XB = 2048
NXB = VIP // XB
CH = 128
NCH = NP // CH
SPB = VP // CH
SSL = 384
NW = 32
CPW = NCH // NW
NBO = (V_OUT + R - 1) // R


def _gather7_body(table_hbm, idxc_hbm, out_hbm, idx_v, acc_v, sem):
    wid = lax.axis_index("s") * 2 + lax.axis_index("c")

    zv = jnp.zeros((16,), jnp.float32)

    def zrow(r, carry):
        acc_v[r, pl.ds(0, 16)] = zv
        acc_v[r, pl.ds(16, 16)] = zv
        return carry

    def chunk(c, carry):
        k = wid * CPW + c
        slot = k + jnp.where(k >= SPB, SSL - SPB, 0)
        pltpu.sync_copy(idxc_hbm.at[slot], idx_v)
        lax.fori_loop(0, CH, zrow, 0)
        cps = [
            pltpu.async_copy(
                table_hbm.at[idx_v.at[pl.ds(j * CH, CH)]], acc_v, sem, add=True
            )
            for j in range(7)
        ]
        for cp in cps:
            cp.wait()
        pltpu.sync_copy(acc_v, out_hbm.at[pl.ds(k * CH, CH)])
        return carry

    lax.fori_loop(0, CPW, chunk, 0)


def _gather7(table, idx_chunks):
    f = functools.partial(
        pl.kernel,
        out_type=jax.ShapeDtypeStruct((NP, C), jnp.float32),
        mesh=plsc.VectorSubcoreMesh(core_axis_name="c", subcore_axis_name="s"),
        scratch_types=[
            pltpu.VMEM((7 * CH,), jnp.int32),
            pltpu.VMEM((CH, C), jnp.float32),
            pltpu.SemaphoreType.DMA,
        ],
        compiler_params=pltpu.CompilerParams(use_tc_tiling_on_sc=False),
    )(_gather7_body)
    return f(table, idx_chunks)


def _xpose_body(x_ref, o_ref):
    x3 = x_ref[0].T.reshape(XB // 4, 4, C)
    o_ref[0] = jnp.concatenate([x3[:, a, :] for a in range(4)], axis=1)


def _xpose(x):
    return pl.pallas_call(
        _xpose_body,
        grid=(B, NXB),
        in_specs=[pl.BlockSpec((1, C, XB), lambda b, i: (b, 0, i))],
        out_specs=pl.BlockSpec((1, XB // 4, 128), lambda b, i: (b, i, 0)),
        out_shape=jax.ShapeDtypeStruct((B, VIP // 4, 128), jnp.float32),
    )(x)


def _idx_body(np_ref, nc_ref, ip_ref, ic_ref):
    b = pl.program_id(0)
    i = pl.program_id(1)

    def chunked(x):
        x = x.reshape(8, 8, 128, 7)
        return x.transpose(0, 1, 3, 2).reshape(448, 128)

    rows = lax.broadcasted_iota(jnp.int32, (448, 128), 0)
    lane = lax.broadcasted_iota(jnp.int32, (448, 128), 1)
    s_blk = rows // 7
    j = rows - 7 * s_blk
    v = i * 8192 + s_blk * 128 + lane
    valid = v < V_OUT
    ip_ref[0, 0] = jnp.where(valid, chunked(np_ref[...]) + b * VIP, 0)
    ic_ref[0, 0] = jnp.where(valid, (chunked(nc_ref[...]) + b * VP) * 7 + j, 0)


def _idx_prep(pool_no, no):
    pad = 48 * 7168 - V_OUT * 7
    npad = jnp.pad(pool_no, (0, pad)).reshape(48, 7168)
    ncad = jnp.pad(no, (0, pad)).reshape(48, 7168)
    out = pl.pallas_call(
        _idx_body,
        grid=(B, 6),
        in_specs=[
            pl.BlockSpec((8, 7168), lambda b, i: (i, 0)),
            pl.BlockSpec((8, 7168), lambda b, i: (i, 0)),
        ],
        out_specs=[
            pl.BlockSpec((1, 1, 448, 128), lambda b, i: (b, i, 0, 0)),
            pl.BlockSpec((1, 1, 448, 128), lambda b, i: (b, i, 0, 0)),
        ],
        out_shape=[
            jax.ShapeDtypeStruct((B, 6, 448, 128), jnp.int32),
            jax.ShapeDtypeStruct((B, 6, 448, 128), jnp.int32),
        ],
    )(npad, ncad)
    return [o.reshape(B * SSL, 7 * CH) for o in out]


def _mm_body(h_ref, w_ref, p_ref):
    p_ref[...] = jnp.dot(h_ref[...], w_ref[...],
                         preferred_element_type=jnp.float32,
                         precision=lax.Precision.HIGHEST)


def _mm(h4, w4):
    return pl.pallas_call(
        _mm_body,
        grid=(NPB,),
        in_specs=[
            pl.BlockSpec((R // 4, 128), lambda i: (i, 0)),
            pl.BlockSpec((128, 4 * 7 * C), lambda i: (0, 0)),
        ],
        out_specs=pl.BlockSpec((R // 4, 4 * 7 * C), lambda i: (i, 0)),
        out_shape=jax.ShapeDtypeStruct((NP // 4, 4 * 7 * C), jnp.float32),
    )(h4, w4)


def _stats_body(c_ref, o_ref):
    i = pl.program_id(1)
    x = c_ref[...]
    r4 = lax.broadcasted_iota(jnp.int32, (R // 4, 128), 0)
    lane = lax.broadcasted_iota(jnp.int32, (R // 4, 128), 1)
    v = i * R + r4 * 4 + lane // C
    xm = jnp.where(v < V_OUT, x, 0.0)
    s1 = jnp.sum(xm, axis=0)[None, :]
    s2 = jnp.sum(xm * xm, axis=0)[None, :]
    blk = jnp.concatenate([s1, s2, jnp.zeros((6, 128), jnp.float32)], axis=0)[None]

    @pl.when(i == 0)
    def _():
        o_ref[...] = blk

    @pl.when(i > 0)
    def _():
        o_ref[...] += blk


def _stats(c4):
    return pl.pallas_call(
        _stats_body,
        grid=(B, NB),
        in_specs=[pl.BlockSpec((R // 4, 128), lambda b, i: (b * NB + i, 0))],
        out_specs=pl.BlockSpec((1, 8, 128), lambda b, i: (b, 0, 0)),
        out_shape=jax.ShapeDtypeStruct((B, 8, 128), jnp.float32),
    )(c4)


def _fold4(x):
    return x[:, 0:32] + x[:, 32:64] + x[:, 64:96] + x[:, 96:128]


def _bn_act4(c_ref, s_ref, gb_ref):
    s = s_ref[0]
    mean = _fold4(s[0:1]) / V_OUT
    var = _fold4(s[1:2]) / V_OUT - mean * mean
    scale = gb_ref[0:1, :C] * lax.rsqrt(var + EPS)
    shift = gb_ref[1:2, :C] - mean * scale
    sc4 = jnp.concatenate([scale] * 4, axis=1)
    sh4 = jnp.concatenate([shift] * 4, axis=1)
    h = c_ref[...] * sc4 + sh4
    return jnp.where(h >= 0, h, NEG * h)


def _bnmm_body(c_ref, s_ref, gb_ref, w_ref, p_ref):
    h = _bn_act4(c_ref, s_ref, gb_ref)
    p_ref[...] = jnp.dot(h, w_ref[...], preferred_element_type=jnp.float32,
                         precision=lax.Precision.HIGHEST)


def _bnmm(c4, s, gb, w4):
    return pl.pallas_call(
        _bnmm_body,
        grid=(B, NB),
        in_specs=[
            pl.BlockSpec((R // 4, 128), lambda b, i: (b * NB + i, 0)),
            pl.BlockSpec((1, 8, 128), lambda b, i: (b, 0, 0)),
            pl.BlockSpec((8, 128), lambda b, i: (0, 0)),
            pl.BlockSpec((128, 4 * 7 * C), lambda b, i: (0, 0)),
        ],
        out_specs=pl.BlockSpec((R // 4, 4 * 7 * C), lambda b, i: (b * NB + i, 0)),
        out_shape=jax.ShapeDtypeStruct((NP // 4, 4 * 7 * C), jnp.float32),
    )(c4, s, gb, w4)


def _final_body(c_ref, s_ref, gb_ref, o_ref):
    h4 = _bn_act4(c_ref, s_ref, gb_ref)
    parts = [h4[:, a * C:(a + 1) * C].T for a in range(4)]
    o_ref[...] = jnp.stack(parts, axis=-1).reshape(C, R)[None]


def _final(c4, s, gb):
    return pl.pallas_call(
        _final_body,
        grid=(B, NBO),
        in_specs=[
            pl.BlockSpec((R // 4, 128), lambda b, i: (b * NB + i, 0)),
            pl.BlockSpec((1, 8, 128), lambda b, i: (b, 0, 0)),
            pl.BlockSpec((8, 128), lambda b, i: (0, 0)),
        ],
        out_specs=pl.BlockSpec((1, C, R), lambda b, i: (b, 0, i)),
        out_shape=jax.ShapeDtypeStruct((B, C, V_OUT), jnp.float32),
    )(c4, s, gb)


def _pack_gb(g, be):
    return jnp.concatenate(
        [
            jnp.pad(g, (0, 128 - C))[None, :],
            jnp.pad(be, (0, 128 - C))[None, :],
            jnp.zeros((6, 128), jnp.float32),
        ],
        axis=0,
    )


def kernel(x, neigh_orders, pool_neigh_orders, W1, b1, g1, be1, W2, b2, g2, be2):
    w1 = W1.reshape(7, C, C).transpose(1, 0, 2).reshape(C, 7 * C) / 7.0
    w2 = W2.reshape(7, C, C).transpose(1, 0, 2).reshape(C, 7 * C)
    eye4 = jnp.eye(4, dtype=jnp.float32)
    w41 = jnp.kron(eye4, w1)
    w42 = jnp.kron(eye4, w2)
    gb1 = _pack_gb(g1, be1)
    gb2 = _pack_gb(g2, be2)

    xt4 = _xpose(x)
    ip, ic = _idx_prep(pool_neigh_orders, neigh_orders)

    h0 = _gather7(xt4.reshape(B * VIP, C), ip)
    p1 = _mm(h0.reshape(NP // 4, 128), w41)
    c1 = _gather7(p1.reshape(NP * 7, C), ic)
    c14 = c1.reshape(NP // 4, 128)
    s1 = _stats(c14)
    p2 = _bnmm(c14, s1, gb1, w42)
    c2 = _gather7(p2.reshape(NP * 7, C), ic)
    c24 = c2.reshape(NP // 4, 128)
    s2 = _stats(c24)
    return _final(c24, s2, gb2)
